# baseline (speedup 1.0000x reference)
import jax
import jax.numpy as jnp
from jax.experimental import pallas as pl
from jax.experimental.pallas import tpu as pltpu


_BLOCK_ROWS = 2048


def _persite_kernel(xa_ref, xb_ref, m_ref, w_ref, b_ref, t_ref, oa_ref, ob_ref):
    w = w_ref[...]
    bb = b_ref[0, 0]
    bl = xa_ref.shape[1]
    ra = jnp.dot(xa_ref[0], w, preferred_element_type=jnp.float32)
    oa_ref[0] = (ra + bb) * m_ref[0, :bl] * jnp.exp(t_ref[:bl])
    rb = jnp.dot(xb_ref[0], w, preferred_element_type=jnp.float32)
    ob_ref[0] = (rb + bb) * m_ref[0, bl:] * jnp.exp(t_ref[bl:])


def kernel(encoded_parents, masks, W, b, log_site_table):
    B, L, D = encoded_parents.shape
    bl = _BLOCK_ROWS
    m3 = masks.reshape(B, L, 1)
    b2 = b.reshape(1, 1)

    oa, ob = pl.pallas_call(
        _persite_kernel,
        grid=(B, L // (2 * bl)),
        in_specs=[
            pl.BlockSpec((1, bl, D), lambda i, j: (i, 2 * j, 0)),
            pl.BlockSpec((1, bl, D), lambda i, j: (i, 2 * j + 1, 0)),
            pl.BlockSpec((1, 2 * bl, 1), lambda i, j: (i, j, 0)),
            pl.BlockSpec((D, 1), lambda i, j: (0, 0)),
            pl.BlockSpec((1, 1), lambda i, j: (0, 0)),
            pl.BlockSpec((2 * bl, 1), lambda i, j: (j, 0)),
        ],
        out_specs=[
            pl.BlockSpec((1, bl, 1), lambda i, j: (i, j, 0)),
            pl.BlockSpec((1, bl, 1), lambda i, j: (i, j, 0)),
        ],
        out_shape=[
            jax.ShapeDtypeStruct((B, L // 2, 1), jnp.float32),
            jax.ShapeDtypeStruct((B, L // 2, 1), jnp.float32),
        ],
        compiler_params=pltpu.CompilerParams(
            dimension_semantics=("parallel", "parallel"),
        ),
    )(encoded_parents, encoded_parents, m3, W, b2, log_site_table)
    nj = L // (2 * bl)
    oa = oa.reshape(B, nj, 1, bl)
    ob = ob.reshape(B, nj, 1, bl)
    return jnp.concatenate([oa, ob], axis=2).reshape(B, L)
